# baseline (device time: 29462 ns/iter reference)
import jax
import jax.numpy as jnp
from jax import lax
from jax.experimental import pallas as pl
from jax.experimental.pallas import tpu as pltpu

N_DEV = 16
M = 512
N = 512
PLANE = 4
ZDIM = 4
QR = M // PLANE
CH = QR // ZDIM


def kernel(A, B):
    def body(a_ref, b_ref, out_ref, partial_ref, prs_buf, q_ref, zrs_buf,
             chunk_ref, qag_buf, ag_buf,
             pa_send, pa_recv, pb_send, pb_recv,
             pc_send, pc_recv, pd_send, pd_recv):
        me = lax.axis_index("i")
        mz = me // PLANE
        mp = me % PLANE

        partial_ref[:, :] = jnp.dot(
            a_ref[:, :].astype(jnp.bfloat16),
            b_ref[:, :].astype(jnp.bfloat16),
            preferred_element_type=jnp.float32,
        ).astype(jnp.bfloat16)

        for d in range(1, PLANE):
            tp = (mp + d) % PLANE
            pltpu.make_async_remote_copy(
                src_ref=partial_ref.at[pl.ds(tp * QR, QR), :],
                dst_ref=prs_buf.at[mp],
                send_sem=pa_send.at[tp],
                recv_sem=pa_recv.at[mp],
                device_id=(mz * PLANE + tp,),
                device_id_type=pl.DeviceIdType.MESH,
            ).start()

        prs_buf[mp] = partial_ref[pl.ds(mp * QR, QR), :]

        for d in range(1, PLANE):
            sp = (mp - d) % PLANE
            pltpu.make_async_remote_copy(
                src_ref=partial_ref.at[pl.ds(0, QR), :],
                dst_ref=prs_buf.at[sp],
                send_sem=pa_send.at[sp],
                recv_sem=pa_recv.at[sp],
                device_id=(sp,),
                device_id_type=pl.DeviceIdType.MESH,
            ).wait_recv()

        q_ref[:, :] = jnp.sum(
            prs_buf[:, :, :].astype(jnp.float32), axis=0
        ).astype(jnp.bfloat16)

        for d in range(1, ZDIM):
            tz = (mz + d) % ZDIM
            pltpu.make_async_remote_copy(
                src_ref=q_ref.at[pl.ds(tz * CH, CH), :],
                dst_ref=zrs_buf.at[mz],
                send_sem=pb_send.at[tz],
                recv_sem=pb_recv.at[mz],
                device_id=(tz * PLANE + mp,),
                device_id_type=pl.DeviceIdType.MESH,
            ).start()

        zrs_buf[mz] = q_ref[pl.ds(mz * CH, CH), :]

        for d in range(1, ZDIM):
            sz = (mz - d) % ZDIM
            pltpu.make_async_remote_copy(
                src_ref=q_ref.at[pl.ds(0, CH), :],
                dst_ref=zrs_buf.at[sz],
                send_sem=pb_send.at[sz],
                recv_sem=pb_recv.at[sz],
                device_id=(sz,),
                device_id_type=pl.DeviceIdType.MESH,
            ).wait_recv()

        z = jnp.sum(zrs_buf[:, :, :].astype(jnp.float32), axis=0)
        silu = z * (1.0 / (1.0 + jnp.exp(-z)))
        chunk_ref[:, :] = silu.astype(jnp.bfloat16)
        qag_buf[pl.ds(mz * CH, CH), :] = chunk_ref[:, :]

        for d in range(1, ZDIM):
            tz = (mz + d) % ZDIM
            pltpu.make_async_remote_copy(
                src_ref=chunk_ref,
                dst_ref=qag_buf.at[pl.ds(mz * CH, CH), :],
                send_sem=pc_send.at[tz],
                recv_sem=pc_recv.at[mz],
                device_id=(tz * PLANE + mp,),
                device_id_type=pl.DeviceIdType.MESH,
            ).start()

        for d in range(1, ZDIM):
            sz = (mz - d) % ZDIM
            pltpu.make_async_remote_copy(
                src_ref=chunk_ref,
                dst_ref=qag_buf.at[pl.ds(sz * CH, CH), :],
                send_sem=pc_send.at[sz],
                recv_sem=pc_recv.at[sz],
                device_id=(sz,),
                device_id_type=pl.DeviceIdType.MESH,
            ).wait_recv()

        ag_buf[pl.ds(mp * QR, QR), :] = qag_buf[:, :]

        for d in range(1, PLANE):
            tp = (mp + d) % PLANE
            pltpu.make_async_remote_copy(
                src_ref=qag_buf,
                dst_ref=ag_buf.at[pl.ds(mp * QR, QR), :],
                send_sem=pd_send.at[tp],
                recv_sem=pd_recv.at[mp],
                device_id=(mz * PLANE + tp,),
                device_id_type=pl.DeviceIdType.MESH,
            ).start()

        for d in range(1, PLANE):
            sp = (mp - d) % PLANE
            pltpu.make_async_remote_copy(
                src_ref=qag_buf,
                dst_ref=ag_buf.at[pl.ds(sp * QR, QR), :],
                send_sem=pd_send.at[sp],
                recv_sem=pd_recv.at[sp],
                device_id=(sp,),
                device_id_type=pl.DeviceIdType.MESH,
            ).wait_recv()

        for d in range(1, PLANE):
            tp = (mp + d) % PLANE
            pltpu.make_async_remote_copy(
                src_ref=partial_ref.at[pl.ds(tp * QR, QR), :],
                dst_ref=prs_buf.at[tp],
                send_sem=pa_send.at[tp],
                recv_sem=pa_recv.at[tp],
                device_id=(tp,),
                device_id_type=pl.DeviceIdType.MESH,
            ).wait_send()
            pltpu.make_async_remote_copy(
                src_ref=qag_buf,
                dst_ref=ag_buf.at[pl.ds(tp * QR, QR), :],
                send_sem=pd_send.at[tp],
                recv_sem=pd_recv.at[tp],
                device_id=(tp,),
                device_id_type=pl.DeviceIdType.MESH,
            ).wait_send()

        for d in range(1, ZDIM):
            tz = (mz + d) % ZDIM
            pltpu.make_async_remote_copy(
                src_ref=q_ref.at[pl.ds(tz * CH, CH), :],
                dst_ref=zrs_buf.at[tz],
                send_sem=pb_send.at[tz],
                recv_sem=pb_recv.at[tz],
                device_id=(tz,),
                device_id_type=pl.DeviceIdType.MESH,
            ).wait_send()
            pltpu.make_async_remote_copy(
                src_ref=chunk_ref,
                dst_ref=qag_buf.at[pl.ds(tz * CH, CH), :],
                send_sem=pc_send.at[tz],
                recv_sem=pc_recv.at[tz],
                device_id=(tz,),
                device_id_type=pl.DeviceIdType.MESH,
            ).wait_send()

        out_ref[:, :] = ag_buf[:, :].astype(jnp.float32)

    return pl.pallas_call(
        body,
        out_shape=jax.ShapeDtypeStruct((M, N), jnp.float32),
        in_specs=[
            pl.BlockSpec(memory_space=pltpu.VMEM),
            pl.BlockSpec(memory_space=pltpu.VMEM),
        ],
        out_specs=pl.BlockSpec(memory_space=pltpu.VMEM),
        scratch_shapes=[
            pltpu.VMEM((M, N), jnp.bfloat16),
            pltpu.VMEM((PLANE, QR, N), jnp.bfloat16),
            pltpu.VMEM((QR, N), jnp.bfloat16),
            pltpu.VMEM((ZDIM, CH, N), jnp.bfloat16),
            pltpu.VMEM((CH, N), jnp.bfloat16),
            pltpu.VMEM((QR, N), jnp.bfloat16),
            pltpu.VMEM((M, N), jnp.bfloat16),
            pltpu.SemaphoreType.DMA((PLANE,)),
            pltpu.SemaphoreType.DMA((PLANE,)),
            pltpu.SemaphoreType.DMA((ZDIM,)),
            pltpu.SemaphoreType.DMA((ZDIM,)),
            pltpu.SemaphoreType.DMA((ZDIM,)),
            pltpu.SemaphoreType.DMA((ZDIM,)),
            pltpu.SemaphoreType.DMA((PLANE,)),
            pltpu.SemaphoreType.DMA((PLANE,)),
        ],
    )(A, B)


# device time: 28775 ns/iter; 1.0239x vs baseline; 1.0239x over previous
import jax
import jax.numpy as jnp
from jax import lax
from jax.experimental import pallas as pl
from jax.experimental.pallas import tpu as pltpu

N_DEV = 16
M = 512
N = 512
CH = M // N_DEV


def kernel(A, B):
    def body(a_ref, b_ref, out_ref, partial_ref, rs_buf, chunk_ref, ag_buf,
             p1_send, p1_recv, p2_send, p2_recv):
        me = lax.axis_index("i")

        partial_ref[:, :] = jnp.dot(
            a_ref[:, :].astype(jnp.bfloat16),
            b_ref[:, :].astype(jnp.bfloat16),
            preferred_element_type=jnp.float32,
        ).astype(jnp.bfloat16)

        for d in range(1, N_DEV):
            j = (me + d) % N_DEV
            rdma = pltpu.make_async_remote_copy(
                src_ref=partial_ref.at[pl.ds(j * CH, CH), :],
                dst_ref=rs_buf.at[me],
                send_sem=p1_send.at[j],
                recv_sem=p1_recv.at[me],
                device_id=(j,),
                device_id_type=pl.DeviceIdType.MESH,
            )
            rdma.start()

        rs_buf[me] = partial_ref[pl.ds(me * CH, CH), :]

        for d in range(1, N_DEV):
            s = (me - d) % N_DEV
            recv = pltpu.make_async_remote_copy(
                src_ref=partial_ref.at[pl.ds(0, CH), :],
                dst_ref=rs_buf.at[s],
                send_sem=p1_send.at[s],
                recv_sem=p1_recv.at[s],
                device_id=(s,),
                device_id_type=pl.DeviceIdType.MESH,
            )
            recv.wait_recv()

        z = jnp.sum(rs_buf[:, :, :].astype(jnp.float32), axis=0)
        silu = z * (1.0 / (1.0 + jnp.exp(-z)))
        chunk_ref[:, :] = silu.astype(jnp.bfloat16)
        ag_buf[pl.ds(me * CH, CH), :] = chunk_ref[:, :]

        mz = me // 4
        mp = me % 4
        p2_order = [((mz + dz) % 4) * 4 + (mp + dp) % 4
                    for dz in (3, 2, 1) for dp in (0, 1, 2, 3)]
        p2_order += [mz * 4 + (mp + dp) % 4 for dp in (1, 2, 3)]
        for j in p2_order:
            rdma = pltpu.make_async_remote_copy(
                src_ref=chunk_ref,
                dst_ref=ag_buf.at[pl.ds(me * CH, CH), :],
                send_sem=p2_send.at[j],
                recv_sem=p2_recv.at[me],
                device_id=(j,),
                device_id_type=pl.DeviceIdType.MESH,
            )
            rdma.start()

        for d in range(1, N_DEV):
            s = (me - d) % N_DEV
            recv = pltpu.make_async_remote_copy(
                src_ref=chunk_ref,
                dst_ref=ag_buf.at[pl.ds(s * CH, CH), :],
                send_sem=p2_send.at[s],
                recv_sem=p2_recv.at[s],
                device_id=(s,),
                device_id_type=pl.DeviceIdType.MESH,
            )
            recv.wait_recv()

        for d in range(1, N_DEV):
            j = (me + d) % N_DEV
            send1 = pltpu.make_async_remote_copy(
                src_ref=partial_ref.at[pl.ds(j * CH, CH), :],
                dst_ref=rs_buf.at[j],
                send_sem=p1_send.at[j],
                recv_sem=p1_recv.at[j],
                device_id=(j,),
                device_id_type=pl.DeviceIdType.MESH,
            )
            send1.wait_send()

            send2 = pltpu.make_async_remote_copy(
                src_ref=chunk_ref,
                dst_ref=ag_buf.at[pl.ds(j * CH, CH), :],
                send_sem=p2_send.at[j],
                recv_sem=p2_recv.at[j],
                device_id=(j,),
                device_id_type=pl.DeviceIdType.MESH,
            )
            send2.wait_send()

        out_ref[:, :] = ag_buf[:, :].astype(jnp.float32)

    return pl.pallas_call(
        body,
        out_shape=jax.ShapeDtypeStruct((M, N), jnp.float32),
        in_specs=[
            pl.BlockSpec(memory_space=pltpu.VMEM),
            pl.BlockSpec(memory_space=pltpu.VMEM),
        ],
        out_specs=pl.BlockSpec(memory_space=pltpu.VMEM),
        scratch_shapes=[
            pltpu.VMEM((M, N), jnp.bfloat16),
            pltpu.VMEM((N_DEV, CH, N), jnp.bfloat16),
            pltpu.VMEM((CH, N), jnp.bfloat16),
            pltpu.VMEM((M, N), jnp.bfloat16),
            pltpu.SemaphoreType.DMA((N_DEV,)),
            pltpu.SemaphoreType.DMA((N_DEV,)),
            pltpu.SemaphoreType.DMA((N_DEV,)),
            pltpu.SemaphoreType.DMA((N_DEV,)),
        ],
    )(A, B)


# device time: 27630 ns/iter; 1.0663x vs baseline; 1.0414x over previous
import jax
import jax.numpy as jnp
from jax import lax
from jax.experimental import pallas as pl
from jax.experimental.pallas import tpu as pltpu

N_DEV = 16
M = 512
N = 512
CH = M // N_DEV
HN = N // 2


def kernel(A, B):
    def body(a_ref, b_ref, out_ref, partial_ref, rs_buf, chunk_ref, ag_buf,
             p1_send, p1_recv, p2_send, p2_recv):
        me = lax.axis_index("i")

        partial_ref[:, :] = jnp.dot(
            a_ref[:, :].astype(jnp.bfloat16),
            b_ref[:, :].astype(jnp.bfloat16),
            preferred_element_type=jnp.float32,
        ).astype(jnp.bfloat16)

        for h in (0, 1):
            for d in range(1, N_DEV):
                j = (me + d) % N_DEV
                pltpu.make_async_remote_copy(
                    src_ref=partial_ref.at[pl.ds(j * CH, CH),
                                           pl.ds(h * HN, HN)],
                    dst_ref=rs_buf.at[me, :, pl.ds(h * HN, HN)],
                    send_sem=p1_send.at[h * N_DEV + j],
                    recv_sem=p1_recv.at[h * N_DEV + me],
                    device_id=(j,),
                    device_id_type=pl.DeviceIdType.MESH,
                ).start()

        rs_buf[me] = partial_ref[pl.ds(me * CH, CH), :]

        for h in (0, 1):
            for d in range(1, N_DEV):
                s = (me - d) % N_DEV
                pltpu.make_async_remote_copy(
                    src_ref=partial_ref.at[pl.ds(0, CH), pl.ds(h * HN, HN)],
                    dst_ref=rs_buf.at[s, :, pl.ds(h * HN, HN)],
                    send_sem=p1_send.at[h * N_DEV + s],
                    recv_sem=p1_recv.at[h * N_DEV + s],
                    device_id=(s,),
                    device_id_type=pl.DeviceIdType.MESH,
                ).wait_recv()

            z = jnp.sum(
                rs_buf[:, :, pl.ds(h * HN, HN)].astype(jnp.float32), axis=0
            )
            silu = z * (1.0 / (1.0 + jnp.exp(-z)))
            chunk_ref[:, pl.ds(h * HN, HN)] = silu.astype(jnp.bfloat16)
            ag_buf[pl.ds(me * CH, CH), pl.ds(h * HN, HN)] = (
                chunk_ref[:, pl.ds(h * HN, HN)]
            )

            for d in range(1, N_DEV):
                j = (me + d) % N_DEV
                pltpu.make_async_remote_copy(
                    src_ref=chunk_ref.at[:, pl.ds(h * HN, HN)],
                    dst_ref=ag_buf.at[pl.ds(me * CH, CH), pl.ds(h * HN, HN)],
                    send_sem=p2_send.at[h * N_DEV + j],
                    recv_sem=p2_recv.at[h * N_DEV + me],
                    device_id=(j,),
                    device_id_type=pl.DeviceIdType.MESH,
                ).start()

        for h in (0, 1):
            for d in range(1, N_DEV):
                s = (me - d) % N_DEV
                pltpu.make_async_remote_copy(
                    src_ref=chunk_ref.at[:, pl.ds(h * HN, HN)],
                    dst_ref=ag_buf.at[pl.ds(s * CH, CH), pl.ds(h * HN, HN)],
                    send_sem=p2_send.at[h * N_DEV + s],
                    recv_sem=p2_recv.at[h * N_DEV + s],
                    device_id=(s,),
                    device_id_type=pl.DeviceIdType.MESH,
                ).wait_recv()

        for h in (0, 1):
            for d in range(1, N_DEV):
                j = (me + d) % N_DEV
                pltpu.make_async_remote_copy(
                    src_ref=partial_ref.at[pl.ds(j * CH, CH),
                                           pl.ds(h * HN, HN)],
                    dst_ref=rs_buf.at[j, :, pl.ds(h * HN, HN)],
                    send_sem=p1_send.at[h * N_DEV + j],
                    recv_sem=p1_recv.at[h * N_DEV + j],
                    device_id=(j,),
                    device_id_type=pl.DeviceIdType.MESH,
                ).wait_send()
                pltpu.make_async_remote_copy(
                    src_ref=chunk_ref.at[:, pl.ds(h * HN, HN)],
                    dst_ref=ag_buf.at[pl.ds(j * CH, CH), pl.ds(h * HN, HN)],
                    send_sem=p2_send.at[h * N_DEV + j],
                    recv_sem=p2_recv.at[h * N_DEV + j],
                    device_id=(j,),
                    device_id_type=pl.DeviceIdType.MESH,
                ).wait_send()

        out_ref[:, :] = ag_buf[:, :].astype(jnp.float32)

    return pl.pallas_call(
        body,
        out_shape=jax.ShapeDtypeStruct((M, N), jnp.float32),
        in_specs=[
            pl.BlockSpec(memory_space=pltpu.VMEM),
            pl.BlockSpec(memory_space=pltpu.VMEM),
        ],
        out_specs=pl.BlockSpec(memory_space=pltpu.VMEM),
        scratch_shapes=[
            pltpu.VMEM((M, N), jnp.bfloat16),
            pltpu.VMEM((N_DEV, CH, N), jnp.bfloat16),
            pltpu.VMEM((CH, N), jnp.bfloat16),
            pltpu.VMEM((M, N), jnp.bfloat16),
            pltpu.SemaphoreType.DMA((2 * N_DEV,)),
            pltpu.SemaphoreType.DMA((2 * N_DEV,)),
            pltpu.SemaphoreType.DMA((2 * N_DEV,)),
            pltpu.SemaphoreType.DMA((2 * N_DEV,)),
        ],
    )(A, B)


# device time: 27059 ns/iter; 1.0888x vs baseline; 1.0211x over previous
import jax
import jax.numpy as jnp
from jax import lax
from jax.experimental import pallas as pl
from jax.experimental.pallas import tpu as pltpu

N_DEV = 16
M = 512
N = 512
CH = M // N_DEV


def kernel(A, B):
    def body(a_ref, b_ref, out_ref, partial_ref, rs_buf, chunk_ref, ag_buf,
             p1_send, p1_recv, p2_send, p2_recv):
        me = lax.axis_index("i")

        partial_ref[:, :] = jnp.dot(
            a_ref[:, :].astype(jnp.bfloat16),
            b_ref[:, :].astype(jnp.bfloat16),
            preferred_element_type=jnp.float32,
        ).astype(jnp.bfloat16)

        for d in range(1, N_DEV):
            j = (me + d) % N_DEV
            rdma = pltpu.make_async_remote_copy(
                src_ref=partial_ref.at[pl.ds(j * CH, CH), :],
                dst_ref=rs_buf.at[me],
                send_sem=p1_send.at[j],
                recv_sem=p1_recv.at[me],
                device_id=(j,),
                device_id_type=pl.DeviceIdType.MESH,
            )
            rdma.start()

        rs_buf[me] = partial_ref[pl.ds(me * CH, CH), :]

        for d in range(1, N_DEV):
            s = (me - d) % N_DEV
            recv = pltpu.make_async_remote_copy(
                src_ref=partial_ref.at[pl.ds(0, CH), :],
                dst_ref=rs_buf.at[s],
                send_sem=p1_send.at[s],
                recv_sem=p1_recv.at[s],
                device_id=(s,),
                device_id_type=pl.DeviceIdType.MESH,
            )
            recv.wait_recv()

        z = jnp.sum(rs_buf[:, :, :].astype(jnp.float32), axis=0)
        silu = z * (1.0 / (1.0 + jnp.exp(-z)))
        chunk_ref[:, :] = silu.astype(jnp.bfloat16)
        out_ref[pl.ds(me * CH, CH), :] = silu

        for d in range(1, N_DEV):
            j = (me + d) % N_DEV
            rdma = pltpu.make_async_remote_copy(
                src_ref=chunk_ref,
                dst_ref=ag_buf.at[pl.ds(me * CH, CH), :],
                send_sem=p2_send.at[j],
                recv_sem=p2_recv.at[me],
                device_id=(j,),
                device_id_type=pl.DeviceIdType.MESH,
            )
            rdma.start()

        for d in range(1, N_DEV):
            s = (me - d) % N_DEV
            recv = pltpu.make_async_remote_copy(
                src_ref=chunk_ref,
                dst_ref=ag_buf.at[pl.ds(s * CH, CH), :],
                send_sem=p2_send.at[s],
                recv_sem=p2_recv.at[s],
                device_id=(s,),
                device_id_type=pl.DeviceIdType.MESH,
            )
            recv.wait_recv()
            out_ref[pl.ds(s * CH, CH), :] = (
                ag_buf[pl.ds(s * CH, CH), :].astype(jnp.float32)
            )

        for d in range(1, N_DEV):
            j = (me + d) % N_DEV
            send1 = pltpu.make_async_remote_copy(
                src_ref=partial_ref.at[pl.ds(j * CH, CH), :],
                dst_ref=rs_buf.at[j],
                send_sem=p1_send.at[j],
                recv_sem=p1_recv.at[j],
                device_id=(j,),
                device_id_type=pl.DeviceIdType.MESH,
            )
            send1.wait_send()

            send2 = pltpu.make_async_remote_copy(
                src_ref=chunk_ref,
                dst_ref=ag_buf.at[pl.ds(j * CH, CH), :],
                send_sem=p2_send.at[j],
                recv_sem=p2_recv.at[j],
                device_id=(j,),
                device_id_type=pl.DeviceIdType.MESH,
            )
            send2.wait_send()

    return pl.pallas_call(
        body,
        out_shape=jax.ShapeDtypeStruct((M, N), jnp.float32),
        in_specs=[
            pl.BlockSpec(memory_space=pltpu.VMEM),
            pl.BlockSpec(memory_space=pltpu.VMEM),
        ],
        out_specs=pl.BlockSpec(memory_space=pltpu.VMEM),
        scratch_shapes=[
            pltpu.VMEM((M, N), jnp.bfloat16),
            pltpu.VMEM((N_DEV, CH, N), jnp.bfloat16),
            pltpu.VMEM((CH, N), jnp.bfloat16),
            pltpu.VMEM((M, N), jnp.bfloat16),
            pltpu.SemaphoreType.DMA((N_DEV,)),
            pltpu.SemaphoreType.DMA((N_DEV,)),
            pltpu.SemaphoreType.DMA((N_DEV,)),
            pltpu.SemaphoreType.DMA((N_DEV,)),
        ],
    )(A, B)
